# baseline (speedup 1.0000x reference)
import functools

import jax
import jax.numpy as jnp
from jax import lax
from jax.experimental import pallas as pl
from jax.experimental.pallas import tpu as pltpu
from jax.experimental.pallas import tpu_sc as plsc

_HEADS = 8
_HC = 16
_HID = 128
_L = 3

_R = 512
_CHUNKS = 6
_CH = 8448
_NPAD = _CH * _CHUNKS
_RPT = _CH // 16
_B = 128
_SROW = 144


_EPT = 26624
_EPAD = 32 * _EPT
_STRIDE = 28032
_STG = 1312


def _f16(v):
    return jnp.full((16,), v, jnp.int32)


def _bucket_body(src_hbm, dst_hbm, bsrc_hbm, bdst_hbm, cnt_hbm,
                 src_s, dst_s, sstage, dstage, cvm):
    cid = lax.axis_index("c")
    tid = lax.axis_index("s")
    w = cid * 16 + tid
    ebase = w * _EPT
    iota = lax.broadcasted_iota(jnp.int32, (16,), 0)
    zi = jnp.zeros((16,), jnp.int32)
    for c in range(_CHUNKS):
        for j in range(_STG // 16):
            plsc.store_scatter(sstage, [_f16(c), iota + j * 16], zi)
            plsc.store_scatter(dstage, [_f16(c), iota + j * 16], zi)

    def macro(mi, carry):
        pltpu.sync_copy(src_hbm.at[pl.ds(pl.multiple_of(ebase + mi * 2048, 8), 2048)], src_s)
        pltpu.sync_copy(dst_hbm.at[pl.ds(pl.multiple_of(ebase + mi * 2048, 8), 2048)], dst_s)

        def grp(g, c2):
            rows = iota + g * 16
            s16 = plsc.load_gather(src_s, [rows])
            d16 = plsc.load_gather(dst_s, [rows])
            out = []
            for c in range(_CHUNKS):
                voff, hoff = c2[2 * c], c2[2 * c + 1]
                m = (d16 >= c * _CH) & (d16 < (c + 1) * _CH)
                mi32 = jnp.where(m, 1, 0)
                pos = voff + plsc.cumsum(mi32) - mi32
                plsc.store_scatter(sstage, [_f16(c), pos], s16, mask=m)
                plsc.store_scatter(dstage, [_f16(c), pos], d16, mask=m)
                voff = voff + jnp.sum(mi32)

                def fl(vh):
                    v, h = vh
                    off = (w * _CHUNKS + c) * _STRIDE + h
                    pltpu.sync_copy(sstage.at[c, pl.ds(0, 1024)],
                                    bsrc_hbm.at[pl.ds(pl.multiple_of(off, 8), 1024)])
                    pltpu.sync_copy(dstage.at[c, pl.ds(0, 1024)],
                                    bdst_hbm.at[pl.ds(pl.multiple_of(off, 8), 1024)])
                    tv = plsc.load_gather(sstage, [_f16(c), iota + 1024])
                    plsc.store_scatter(sstage, [_f16(c), iota], tv)
                    tv = plsc.load_gather(dstage, [_f16(c), iota + 1024])
                    plsc.store_scatter(dstage, [_f16(c), iota], tv)
                    return (v - 1024, h + 1024)

                voff, hoff = lax.cond(voff >= 1024, fl, lambda vh: vh,
                                      (voff, hoff))
                out += [voff, hoff]
            return tuple(out)

        return lax.fori_loop(0, 128, grp, carry)

    carry = lax.fori_loop(0, _EPT // 2048, macro, (jnp.int32(0),) * (2 * _CHUNKS))
    cvec = jnp.zeros((16,), jnp.int32)
    for c in range(_CHUNKS):
        voff, hoff = carry[2 * c], carry[2 * c + 1]
        off = (w * _CHUNKS + c) * _STRIDE + hoff
        pltpu.sync_copy(sstage.at[c, pl.ds(0, 1312)],
                        bsrc_hbm.at[pl.ds(pl.multiple_of(off, 8), 1312)])
        pltpu.sync_copy(dstage.at[c, pl.ds(0, 1312)],
                        bdst_hbm.at[pl.ds(pl.multiple_of(off, 8), 1312)])
        cvec = jnp.where(iota == c, voff + hoff, cvec)
    cvm[...] = cvec
    pltpu.sync_copy(cvm, cnt_hbm.at[pl.ds(pl.multiple_of(w * 16, 8), 16)])


def _sc_bucket(src, dst):
    run = pl.kernel(
        _bucket_body,
        out_type=[jax.ShapeDtypeStruct((32 * _CHUNKS * _STRIDE,), jnp.int32),
                  jax.ShapeDtypeStruct((32 * _CHUNKS * _STRIDE,), jnp.int32),
                  jax.ShapeDtypeStruct((512,), jnp.int32)],
        mesh=plsc.VectorSubcoreMesh(core_axis_name="c", subcore_axis_name="s"),
        compiler_params=pltpu.CompilerParams(use_tc_tiling_on_sc=False,
                                             needs_layout_passes=False),
        scratch_types=[
            pltpu.VMEM((2048,), jnp.int32),
            pltpu.VMEM((2048,), jnp.int32),
            pltpu.VMEM((_CHUNKS, _STG), jnp.int32),
            pltpu.VMEM((_CHUNKS, _STG), jnp.int32),
            pltpu.VMEM((16,), jnp.int32),
        ],
    )
    return run(src, dst)


def _sc_body(bsrc_hbm, bdst_hbm, cnt_hbm, s_hbm, ad_hbm, accinit_hbm, out_hbm,
             cnts_vm,
             srcb_a, dstb_a, dreli_a, svm_a, advm_a,
             srcb_b, dstb_b, dreli_b, svm_b, advm_b,
             acc_sh, sem_ea, sem_eb, sem_ga, sem_gb, sem_sa, sem_sb):
    cid = lax.axis_index("c")
    tid = lax.axis_index("s")
    iota = lax.broadcasted_iota(jnp.int32, (16,), 0)
    pltpu.sync_copy(cnt_hbm, cnts_vm)

    def payload(svm, advm):
        def grp(g, c2):
            rows = iota + g * 16
            for h in range(_HEADS):
                asv = plsc.load_gather(svm, [rows, _f16(128 + h)])
                adv = plsc.load_gather(advm, [rows, _f16(h)])
                al = asv + adv
                al = jnp.where(al >= 0, al, 0.2 * al)
                wgt = jnp.exp(al)
                plsc.store_scatter(svm, [rows, _f16(128 + h)], wgt)
                for cc in range(_HC):
                    col = h * _HC + cc
                    xv = plsc.load_gather(svm, [rows, _f16(col)])
                    plsc.store_scatter(svm, [rows, _f16(col)], xv * wgt)
            return c2

        lax.fori_loop(0, _B // 16, grp, 0)

    def chunk_step(k, kcarry):
        chunk = cid * (_CHUNKS // 2) + k
        base = chunk * _CH
        pltpu.sync_copy(accinit_hbm.at[pl.ds(base + tid * _RPT, _RPT)],
                        acc_sh.at[pl.ds(tid * _RPT, _RPT)])
        plsc.subcore_barrier()

        for wi in range(2):
            w = tid + wi * 16
            rowoff = (w * _CHUNKS + chunk) * _STRIDE
            cv = cnts_vm[pl.ds(pl.multiple_of(w * 16, 8), 16)]
            cnt = jnp.sum(jnp.where(iota == chunk, cv, 0))
            nblk = (cnt + _B - 1) // _B
            nstep = (nblk + 1) // 2

            def eload(bi, srcb, dstb, sem):
                off = pl.multiple_of(rowoff + bi * _B, 8)
                pltpu.async_copy(bsrc_hbm.at[pl.ds(off, _B)], srcb, sem)
                pltpu.async_copy(bdst_hbm.at[pl.ds(off, _B)], dstb, sem)

            def ewait(srcb, dstb, sem):
                pltpu.make_async_copy(
                    bsrc_hbm.at[pl.ds(0, _B)], srcb, sem).wait()
                pltpu.make_async_copy(
                    bdst_hbm.at[pl.ds(0, _B)], dstb, sem).wait()

            def half(j, bi, srcb, dstb, dreli, svm, advm,
                     sem_e, sem_g, sem_s, pf_bi, srcb_o, dstb_o, sem_eo):
                ewait(srcb, dstb, sem_e)

                @pl.when(j > 0)
                def _():
                    pltpu.make_async_copy(svm, acc_sh.at[dreli], sem_s).wait()

                for i in range(4):
                    pltpu.async_copy(
                        s_hbm.at[srcb.at[pl.ds(32 * i, 32)]],
                        svm.at[pl.ds(32 * i, 32)], sem_g)
                ga = pltpu.async_copy(ad_hbm.at[dstb], advm, sem_g)
                eload(pf_bi, srcb_o, dstb_o, sem_eo)
                for g in range(_B // 16):
                    d16 = dstb[pl.ds(g * 16, 16)]
                    pos = bi * _B + g * 16 + iota
                    dreli[pl.ds(g * 16, 16)] = jnp.where(pos < cnt,
                                                         d16 - base, _CH)
                for i in range(4):
                    pltpu.make_async_copy(
                        s_hbm.at[srcb.at[pl.ds(32 * i, 32)]],
                        svm.at[pl.ds(32 * i, 32)], sem_g).wait()
                ga.wait()
                payload(svm, advm)
                pltpu.async_copy(svm, acc_sh.at[dreli], sem_s, add=True)

            eload(0, srcb_a, dstb_a, sem_ea)

            def body(j, carry):
                half(j, 2 * j, srcb_a, dstb_a, dreli_a, svm_a, advm_a,
                     sem_ea, sem_ga, sem_sa,
                     2 * j + 1, srcb_b, dstb_b, sem_eb)
                half(j, 2 * j + 1, srcb_b, dstb_b, dreli_b, svm_b, advm_b,
                     sem_eb, sem_gb, sem_sb,
                     2 * j + 2, srcb_a, dstb_a, sem_ea)
                return carry

            lax.fori_loop(0, nstep, body, 0)
            ewait(srcb_a, dstb_a, sem_ea)

            @pl.when(nstep > 0)
            def _():
                pltpu.make_async_copy(svm_a, acc_sh.at[dreli_a], sem_sa).wait()
                pltpu.make_async_copy(svm_b, acc_sh.at[dreli_b], sem_sb).wait()

        plsc.subcore_barrier()
        pltpu.sync_copy(acc_sh.at[pl.ds(tid * _RPT, _RPT)],
                        out_hbm.at[pl.ds(base + tid * _RPT, _RPT)])
        plsc.subcore_barrier()
        return kcarry

    lax.fori_loop(0, _CHUNKS // 2, chunk_step, 0)


def _sc_aggregate(bsrc, bdst, cnts, s_tab, ad_tab, acc_init):
    run = pl.kernel(
        _sc_body,
        out_type=jax.ShapeDtypeStruct((_NPAD, _SROW), jnp.float32),
        mesh=plsc.VectorSubcoreMesh(core_axis_name="c", subcore_axis_name="s"),
        compiler_params=pltpu.CompilerParams(use_tc_tiling_on_sc=False,
                                             needs_layout_passes=False),
        scratch_types=(
            [pltpu.VMEM((512,), jnp.int32)]
            + 2 * [pltpu.VMEM((_B,), jnp.int32),
                   pltpu.VMEM((_B,), jnp.int32),
                   pltpu.VMEM((_B,), jnp.int32),
                   pltpu.VMEM((_B, _SROW), jnp.float32),
                   pltpu.VMEM((_B, 16), jnp.float32)]
            + [pltpu.VMEM_SHARED((_CH + 16, _SROW), jnp.float32)]
            + 6 * [pltpu.SemaphoreType.DMA]
        ),
    )
    return run(bsrc, bdst, cnts, s_tab, ad_tab, acc_init)


def _prep_common(y, w_ref, ab_ref, rep_ref, s_ref, ad_ref, acc_ref):
    xp = jnp.dot(y, w_ref[...], preferred_element_type=jnp.float32)
    both = jnp.dot(xp, ab_ref[...], preferred_element_type=jnp.float32)
    a_s = both[:, :8]
    a_d = both[:, 8:]
    t = a_s + a_d
    w_self = jnp.exp(jnp.where(t >= 0, t, 0.2 * t))
    wrep = jnp.dot(w_self, rep_ref[...], preferred_element_type=jnp.float32)
    z8 = jnp.zeros((y.shape[0], 8), jnp.float32)
    s_ref[...] = jnp.concatenate([xp, a_s, z8], axis=1)
    ad_ref[...] = jnp.concatenate([a_d, z8], axis=1)
    acc_ref[...] = jnp.concatenate([xp * wrep, w_self, z8], axis=1)


def _prep0_body(atom_ref, wa_ref, ba_ref, w_ref, ab_ref, rep_ref,
                s_ref, ad_ref, acc_ref):
    y = jnp.dot(atom_ref[...], wa_ref[...],
                preferred_element_type=jnp.float32) + ba_ref[...]
    _prep_common(y, w_ref, ab_ref, rep_ref, s_ref, ad_ref, acc_ref)


def _prepl_body(n, x_ref, sums_ref, bng_ref, bnb_ref, w_ref, ab_ref, rep_ref,
                s_ref, ad_ref, acc_ref):
    mu = sums_ref[0:1, :] / n
    var = sums_ref[1:2, :] / n - mu * mu
    rstd = lax.rsqrt(var + 1e-5)
    y = (x_ref[...] - mu) * rstd * bng_ref[...] + bnb_ref[...]
    y = jnp.maximum(y, 0.0)
    _prep_common(y, w_ref, ab_ref, rep_ref, s_ref, ad_ref, acc_ref)


def _fin_body(n, acc_ref, rep_ref, gatb_ref, x_ref, sums_ref):
    acc = acc_ref[...]
    dn = jnp.dot(acc[:, 128:136], rep_ref[...],
                 preferred_element_type=jnp.float32)
    x = acc[:, :128] / dn + gatb_ref[...]
    x_ref[...] = x
    rows = pl.program_id(0) * _R + lax.broadcasted_iota(jnp.int32, (_R, 1), 0)
    xm = jnp.where(rows < n, x, 0.0)
    blk = jnp.concatenate(
        [jnp.sum(xm, axis=0, keepdims=True),
         jnp.sum(xm * xm, axis=0, keepdims=True),
         jnp.zeros((6, 128), jnp.float32)], axis=0)

    @pl.when(pl.program_id(0) == 0)
    def _():
        sums_ref[...] = blk

    @pl.when(pl.program_id(0) != 0)
    def _():
        sums_ref[...] += blk


def _colsum_body(n, x_ref, sums_ref, bng_ref, bnb_ref, ysum_ref):
    mu = sums_ref[0:1, :] / n
    var = sums_ref[1:2, :] / n - mu * mu
    rstd = lax.rsqrt(var + 1e-5)
    y = (x_ref[...] - mu) * rstd * bng_ref[...] + bnb_ref[...]
    y = jnp.maximum(y, 0.0)
    rows = pl.program_id(0) * _R + lax.broadcasted_iota(jnp.int32, (_R, 1), 0)
    ym = jnp.where(rows < n, y, 0.0)
    blk = jnp.concatenate(
        [jnp.sum(ym, axis=0, keepdims=True),
         jnp.zeros((7, 128), jnp.float32)], axis=0)

    @pl.when(pl.program_id(0) == 0)
    def _():
        ysum_ref[...] = blk

    @pl.when(pl.program_id(0) != 0)
    def _():
        ysum_ref[...] += blk


def _head_body(n, ysum_ref, w1_ref, b1_ref, w2_ref, b2_ref, o_ref):
    mean = ysum_ref[0:1, :] / n
    h = jnp.maximum(jnp.dot(mean, w1_ref[...],
                            preferred_element_type=jnp.float32) + b1_ref[...],
                    0.0)
    o_ref[...] = jnp.dot(h, w2_ref[...],
                         preferred_element_type=jnp.float32) + b2_ref[...]


def _row_blocked_call(body, n_extra_in, out_specs, out_shapes):
    grid = _NPAD // _R
    return body, grid


_FULL = lambda *shape: pl.BlockSpec(shape, lambda i: (0,) * len(shape))


def _prep_call(body, x_first, extras):
    grid = _NPAD // _R
    in_specs = [pl.BlockSpec((_R, x_first.shape[1]), lambda i: (i, 0))]
    in_specs += [_FULL(*e.shape) for e in extras]
    out = pl.pallas_call(
        body,
        grid=(grid,),
        in_specs=in_specs,
        out_specs=[
            pl.BlockSpec((_R, _SROW), lambda i: (i, 0)),
            pl.BlockSpec((_R, 16), lambda i: (i, 0)),
            pl.BlockSpec((_R, _SROW), lambda i: (i, 0)),
        ],
        out_shape=[
            jax.ShapeDtypeStruct((_NPAD, _SROW), jnp.float32),
            jax.ShapeDtypeStruct((_NPAD, 16), jnp.float32),
            jax.ShapeDtypeStruct((_NPAD, _SROW), jnp.float32),
        ],
    )(x_first, *extras)
    return out


def kernel(atom_features, bond_features, edge_index, W_atom, b_atom, gat_W,
           att_src, att_dst, gat_b, bn_g, bn_b, W1, b1, W2, b2):
    n = atom_features.shape[0]
    nf = float(n)
    e = edge_index.shape[1]

    atom_p = jnp.pad(atom_features, ((0, _NPAD - n), (0, 128 - atom_features.shape[1])))
    wa_p = jnp.pad(W_atom, ((0, 128 - W_atom.shape[0]), (0, 0)))
    src = jnp.pad(edge_index[0], (0, _EPAD - e))
    dst = jnp.pad(edge_index[1], (0, _EPAD - e), constant_values=0x3F000000)
    bsrc, bdst, cnts = _sc_bucket(src, dst)

    rep = jnp.repeat(jnp.eye(8, dtype=jnp.float32), _HC, axis=1)
    hid_ids = jnp.repeat(jnp.arange(8), _HC)
    cols = jnp.arange(128)

    def make_ab(l):
        ab = jnp.zeros((128, 16), jnp.float32)
        ab = ab.at[cols, hid_ids].set(att_src[l].reshape(128))
        ab = ab.at[cols, 8 + hid_ids].set(att_dst[l].reshape(128))
        return ab

    s_tab, ad_tab, acc_init = _prep_call(
        _prep0_body, atom_p,
        [wa_p, b_atom[None, :], gat_W[0], make_ab(0), rep])

    grid = _NPAD // _R
    x = None
    sums = None
    for l in range(_L):
        acc = _sc_aggregate(bsrc, bdst, cnts, s_tab, ad_tab, acc_init)
        x, sums = pl.pallas_call(
            functools.partial(_fin_body, nf),
            grid=(grid,),
            in_specs=[pl.BlockSpec((_R, _SROW), lambda i: (i, 0)),
                      _FULL(8, 128), _FULL(1, 128)],
            out_specs=[pl.BlockSpec((_R, 128), lambda i: (i, 0)),
                       _FULL(8, 128)],
            out_shape=[jax.ShapeDtypeStruct((_NPAD, 128), jnp.float32),
                       jax.ShapeDtypeStruct((8, 128), jnp.float32)],
        )(acc, rep, gat_b[l][None, :])
        if l < _L - 1:
            s_tab, ad_tab, acc_init = _prep_call(
                functools.partial(_prepl_body, nf), x,
                [sums, bn_g[l][None, :], bn_b[l][None, :],
                 gat_W[l + 1], make_ab(l + 1), rep])

    ysum = pl.pallas_call(
        functools.partial(_colsum_body, nf),
        grid=(grid,),
        in_specs=[pl.BlockSpec((_R, 128), lambda i: (i, 0)),
                  _FULL(8, 128), _FULL(1, 128), _FULL(1, 128)],
        out_specs=_FULL(8, 128),
        out_shape=jax.ShapeDtypeStruct((8, 128), jnp.float32),
    )(x, sums, bn_g[_L - 1][None, :], bn_b[_L - 1][None, :])

    out = pl.pallas_call(
        functools.partial(_head_body, nf),
        grid=(1,),
        in_specs=[_FULL(8, 128), _FULL(*W1.shape), _FULL(1, W1.shape[1]),
                  _FULL(*W2.shape), _FULL(1, W2.shape[1])],
        out_specs=_FULL(1, W2.shape[1]),
        out_shape=jax.ShapeDtypeStruct((1, W2.shape[1]), jnp.float32),
    )(ysum, W1, b1[None, :], W2, b2[None, :])
    return out


# speedup vs baseline: 1.2908x; 1.2908x over previous
import functools

import jax
import jax.numpy as jnp
from jax import lax
from jax.experimental import pallas as pl
from jax.experimental.pallas import tpu as pltpu
from jax.experimental.pallas import tpu_sc as plsc

_HEADS = 8
_HC = 16
_HID = 128
_L = 3

_R = 512
_CHUNKS = 6
_CH = 8448
_NPAD = _CH * _CHUNKS
_RPT = _CH // 16
_B = 128
_SROW = 144


_EPT = 26624
_EPAD = 32 * _EPT
_STRIDE = 28032
_STG = 1312


def _f16(v):
    return jnp.full((16,), v, jnp.int32)


def _bucket_body(src_hbm, dst_hbm, bsrc_hbm, bdst_hbm, cnt_hbm,
                 src_s, dst_s, sstage, dstage, cvm):
    cid = lax.axis_index("c")
    tid = lax.axis_index("s")
    w = cid * 16 + tid
    ebase = w * _EPT
    iota = lax.broadcasted_iota(jnp.int32, (16,), 0)
    zi = jnp.zeros((16,), jnp.int32)
    for c in range(_CHUNKS):
        for j in range(_STG // 16):
            plsc.store_scatter(sstage, [_f16(c), iota + j * 16], zi)
            plsc.store_scatter(dstage, [_f16(c), iota + j * 16], zi)

    def macro(mi, carry):
        pltpu.sync_copy(src_hbm.at[pl.ds(pl.multiple_of(ebase + mi * 2048, 8), 2048)], src_s)
        pltpu.sync_copy(dst_hbm.at[pl.ds(pl.multiple_of(ebase + mi * 2048, 8), 2048)], dst_s)

        def grp(g, c2):
            rows = iota + g * 16
            s16 = plsc.load_gather(src_s, [rows])
            d16 = plsc.load_gather(dst_s, [rows])
            out = []
            for c in range(_CHUNKS):
                voff, hoff = c2[2 * c], c2[2 * c + 1]
                m = (d16 >= c * _CH) & (d16 < (c + 1) * _CH)
                mi32 = jnp.where(m, 1, 0)
                pos = voff + plsc.cumsum(mi32) - mi32
                plsc.store_scatter(sstage, [_f16(c), pos], s16, mask=m)
                plsc.store_scatter(dstage, [_f16(c), pos], d16, mask=m)
                voff = voff + jnp.sum(mi32)

                def fl(vh):
                    v, h = vh
                    off = (w * _CHUNKS + c) * _STRIDE + h
                    pltpu.sync_copy(sstage.at[c, pl.ds(0, 1024)],
                                    bsrc_hbm.at[pl.ds(pl.multiple_of(off, 8), 1024)])
                    pltpu.sync_copy(dstage.at[c, pl.ds(0, 1024)],
                                    bdst_hbm.at[pl.ds(pl.multiple_of(off, 8), 1024)])
                    tv = plsc.load_gather(sstage, [_f16(c), iota + 1024])
                    plsc.store_scatter(sstage, [_f16(c), iota], tv)
                    tv = plsc.load_gather(dstage, [_f16(c), iota + 1024])
                    plsc.store_scatter(dstage, [_f16(c), iota], tv)
                    return (v - 1024, h + 1024)

                voff, hoff = lax.cond(voff >= 1024, fl, lambda vh: vh,
                                      (voff, hoff))
                out += [voff, hoff]
            return tuple(out)

        return lax.fori_loop(0, 128, grp, carry)

    carry = lax.fori_loop(0, _EPT // 2048, macro, (jnp.int32(0),) * (2 * _CHUNKS))
    cvec = jnp.zeros((16,), jnp.int32)
    for c in range(_CHUNKS):
        voff, hoff = carry[2 * c], carry[2 * c + 1]
        off = (w * _CHUNKS + c) * _STRIDE + hoff
        pltpu.sync_copy(sstage.at[c, pl.ds(0, 1312)],
                        bsrc_hbm.at[pl.ds(pl.multiple_of(off, 8), 1312)])
        pltpu.sync_copy(dstage.at[c, pl.ds(0, 1312)],
                        bdst_hbm.at[pl.ds(pl.multiple_of(off, 8), 1312)])
        cvec = jnp.where(iota == c, voff + hoff, cvec)
    cvm[...] = cvec
    pltpu.sync_copy(cvm, cnt_hbm.at[pl.ds(pl.multiple_of(w * 16, 8), 16)])


def _sc_bucket(src, dst):
    run = pl.kernel(
        _bucket_body,
        out_type=[jax.ShapeDtypeStruct((32 * _CHUNKS * _STRIDE,), jnp.int32),
                  jax.ShapeDtypeStruct((32 * _CHUNKS * _STRIDE,), jnp.int32),
                  jax.ShapeDtypeStruct((512,), jnp.int32)],
        mesh=plsc.VectorSubcoreMesh(core_axis_name="c", subcore_axis_name="s"),
        compiler_params=pltpu.CompilerParams(use_tc_tiling_on_sc=False,
                                             needs_layout_passes=False),
        scratch_types=[
            pltpu.VMEM((2048,), jnp.int32),
            pltpu.VMEM((2048,), jnp.int32),
            pltpu.VMEM((_CHUNKS, _STG), jnp.int32),
            pltpu.VMEM((_CHUNKS, _STG), jnp.int32),
            pltpu.VMEM((16,), jnp.int32),
        ],
    )
    return run(src, dst)


def _sc_body(bsrc_hbm, bdst_hbm, cnt_hbm, s_hbm, ad_hbm, accinit_hbm, out_hbm,
             cnts_vm,
             srcb_a, dstb_a, dreli_a, svm_a, advm_a,
             srcb_b, dstb_b, dreli_b, svm_b, advm_b,
             acc_sh, sem_ea, sem_eb, sem_ga, sem_gb, sem_sa, sem_sb):
    cid = lax.axis_index("c")
    tid = lax.axis_index("s")
    iota = lax.broadcasted_iota(jnp.int32, (16,), 0)
    pltpu.sync_copy(cnt_hbm, cnts_vm)

    def payload(svm, advm):
        def edge(e, c2):
            rowv = iota * 0 + e
            colw = iota + 128
            as16 = plsc.load_gather(svm, [rowv, colw])
            ad16 = plsc.load_gather(advm, [rowv, iota])
            al = as16 + ad16
            al = jnp.where(al >= 0, al, 0.2 * al)
            w16 = jnp.exp(al)
            plsc.store_scatter(svm, [rowv, colw], w16)
            for h in range(_HEADS):
                ws = jnp.broadcast_to(w16[h], (16,))
                cols = iota + h * _HC
                xv = plsc.load_gather(svm, [rowv, cols])
                plsc.store_scatter(svm, [rowv, cols], xv * ws)
            return c2

        lax.fori_loop(0, _B, edge, 0)

    def chunk_step(k, kcarry):
        chunk = cid * (_CHUNKS // 2) + k
        base = chunk * _CH
        pltpu.sync_copy(accinit_hbm.at[pl.ds(base + tid * _RPT, _RPT)],
                        acc_sh.at[pl.ds(tid * _RPT, _RPT)])
        plsc.subcore_barrier()

        for wi in range(2):
            w = tid + wi * 16
            rowoff = (w * _CHUNKS + chunk) * _STRIDE
            cv = cnts_vm[pl.ds(pl.multiple_of(w * 16, 8), 16)]
            cnt = jnp.sum(jnp.where(iota == chunk, cv, 0))
            nblk = (cnt + _B - 1) // _B
            nstep = (nblk + 1) // 2

            def eload(bi, srcb, dstb, sem):
                off = pl.multiple_of(rowoff + bi * _B, 8)
                pltpu.async_copy(bsrc_hbm.at[pl.ds(off, _B)], srcb, sem)
                pltpu.async_copy(bdst_hbm.at[pl.ds(off, _B)], dstb, sem)

            def ewait(srcb, dstb, sem):
                pltpu.make_async_copy(
                    bsrc_hbm.at[pl.ds(0, _B)], srcb, sem).wait()
                pltpu.make_async_copy(
                    bdst_hbm.at[pl.ds(0, _B)], dstb, sem).wait()

            def half(j, bi, srcb, dstb, dreli, svm, advm,
                     sem_e, sem_g, sem_s, pf_bi, srcb_o, dstb_o, sem_eo):
                ewait(srcb, dstb, sem_e)

                @pl.when(j > 0)
                def _():
                    pltpu.make_async_copy(svm, acc_sh.at[dreli], sem_s).wait()

                gs = pltpu.async_copy(s_hbm.at[srcb], svm, sem_g)
                ga = pltpu.async_copy(ad_hbm.at[dstb], advm, sem_g)
                eload(pf_bi, srcb_o, dstb_o, sem_eo)
                for g in range(_B // 16):
                    d16 = dstb[pl.ds(g * 16, 16)]
                    pos = bi * _B + g * 16 + iota
                    dreli[pl.ds(g * 16, 16)] = jnp.where(pos < cnt,
                                                         d16 - base, _CH)
                gs.wait()
                ga.wait()
                payload(svm, advm)
                pltpu.async_copy(svm, acc_sh.at[dreli], sem_s, add=True)

            eload(0, srcb_a, dstb_a, sem_ea)

            def body(j, carry):
                half(j, 2 * j, srcb_a, dstb_a, dreli_a, svm_a, advm_a,
                     sem_ea, sem_ga, sem_sa,
                     2 * j + 1, srcb_b, dstb_b, sem_eb)
                half(j, 2 * j + 1, srcb_b, dstb_b, dreli_b, svm_b, advm_b,
                     sem_eb, sem_gb, sem_sb,
                     2 * j + 2, srcb_a, dstb_a, sem_ea)
                return carry

            lax.fori_loop(0, nstep, body, 0)
            ewait(srcb_a, dstb_a, sem_ea)

            @pl.when(nstep > 0)
            def _():
                pltpu.make_async_copy(svm_a, acc_sh.at[dreli_a], sem_sa).wait()
                pltpu.make_async_copy(svm_b, acc_sh.at[dreli_b], sem_sb).wait()

        plsc.subcore_barrier()
        pltpu.sync_copy(acc_sh.at[pl.ds(tid * _RPT, _RPT)],
                        out_hbm.at[pl.ds(base + tid * _RPT, _RPT)])
        plsc.subcore_barrier()
        return kcarry

    lax.fori_loop(0, _CHUNKS // 2, chunk_step, 0)


def _sc_aggregate(bsrc, bdst, cnts, s_tab, ad_tab, acc_init):
    run = pl.kernel(
        _sc_body,
        out_type=jax.ShapeDtypeStruct((_NPAD, _SROW), jnp.float32),
        mesh=plsc.VectorSubcoreMesh(core_axis_name="c", subcore_axis_name="s"),
        compiler_params=pltpu.CompilerParams(use_tc_tiling_on_sc=False,
                                             needs_layout_passes=False),
        scratch_types=(
            [pltpu.VMEM((512,), jnp.int32)]
            + 2 * [pltpu.VMEM((_B,), jnp.int32),
                   pltpu.VMEM((_B,), jnp.int32),
                   pltpu.VMEM((_B,), jnp.int32),
                   pltpu.VMEM((_B, _SROW), jnp.float32),
                   pltpu.VMEM((_B, 16), jnp.float32)]
            + [pltpu.VMEM_SHARED((_CH + 16, _SROW), jnp.float32)]
            + 6 * [pltpu.SemaphoreType.DMA]
        ),
    )
    return run(bsrc, bdst, cnts, s_tab, ad_tab, acc_init)


def _prep_common(y, w_ref, ab_ref, rep_ref, s_ref, ad_ref, acc_ref):
    xp = jnp.dot(y, w_ref[...], preferred_element_type=jnp.float32)
    both = jnp.dot(xp, ab_ref[...], preferred_element_type=jnp.float32)
    a_s = both[:, :8]
    a_d = both[:, 8:]
    t = a_s + a_d
    w_self = jnp.exp(jnp.where(t >= 0, t, 0.2 * t))
    wrep = jnp.dot(w_self, rep_ref[...], preferred_element_type=jnp.float32)
    z8 = jnp.zeros((y.shape[0], 8), jnp.float32)
    s_ref[...] = jnp.concatenate([xp, a_s, z8], axis=1)
    ad_ref[...] = jnp.concatenate([a_d, z8], axis=1)
    acc_ref[...] = jnp.concatenate([xp * wrep, w_self, z8], axis=1)


def _prep0_body(atom_ref, wa_ref, ba_ref, w_ref, ab_ref, rep_ref,
                s_ref, ad_ref, acc_ref):
    y = jnp.dot(atom_ref[...], wa_ref[...],
                preferred_element_type=jnp.float32) + ba_ref[...]
    _prep_common(y, w_ref, ab_ref, rep_ref, s_ref, ad_ref, acc_ref)


def _prepl_body(n, x_ref, sums_ref, bng_ref, bnb_ref, w_ref, ab_ref, rep_ref,
                s_ref, ad_ref, acc_ref):
    mu = sums_ref[0:1, :] / n
    var = sums_ref[1:2, :] / n - mu * mu
    rstd = lax.rsqrt(var + 1e-5)
    y = (x_ref[...] - mu) * rstd * bng_ref[...] + bnb_ref[...]
    y = jnp.maximum(y, 0.0)
    _prep_common(y, w_ref, ab_ref, rep_ref, s_ref, ad_ref, acc_ref)


def _fin_body(n, acc_ref, rep_ref, gatb_ref, x_ref, sums_ref):
    acc = acc_ref[...]
    dn = jnp.dot(acc[:, 128:136], rep_ref[...],
                 preferred_element_type=jnp.float32)
    x = acc[:, :128] / dn + gatb_ref[...]
    x_ref[...] = x
    rows = pl.program_id(0) * _R + lax.broadcasted_iota(jnp.int32, (_R, 1), 0)
    xm = jnp.where(rows < n, x, 0.0)
    blk = jnp.concatenate(
        [jnp.sum(xm, axis=0, keepdims=True),
         jnp.sum(xm * xm, axis=0, keepdims=True),
         jnp.zeros((6, 128), jnp.float32)], axis=0)

    @pl.when(pl.program_id(0) == 0)
    def _():
        sums_ref[...] = blk

    @pl.when(pl.program_id(0) != 0)
    def _():
        sums_ref[...] += blk


def _colsum_body(n, x_ref, sums_ref, bng_ref, bnb_ref, ysum_ref):
    mu = sums_ref[0:1, :] / n
    var = sums_ref[1:2, :] / n - mu * mu
    rstd = lax.rsqrt(var + 1e-5)
    y = (x_ref[...] - mu) * rstd * bng_ref[...] + bnb_ref[...]
    y = jnp.maximum(y, 0.0)
    rows = pl.program_id(0) * _R + lax.broadcasted_iota(jnp.int32, (_R, 1), 0)
    ym = jnp.where(rows < n, y, 0.0)
    blk = jnp.concatenate(
        [jnp.sum(ym, axis=0, keepdims=True),
         jnp.zeros((7, 128), jnp.float32)], axis=0)

    @pl.when(pl.program_id(0) == 0)
    def _():
        ysum_ref[...] = blk

    @pl.when(pl.program_id(0) != 0)
    def _():
        ysum_ref[...] += blk


def _head_body(n, ysum_ref, w1_ref, b1_ref, w2_ref, b2_ref, o_ref):
    mean = ysum_ref[0:1, :] / n
    h = jnp.maximum(jnp.dot(mean, w1_ref[...],
                            preferred_element_type=jnp.float32) + b1_ref[...],
                    0.0)
    o_ref[...] = jnp.dot(h, w2_ref[...],
                         preferred_element_type=jnp.float32) + b2_ref[...]


def _row_blocked_call(body, n_extra_in, out_specs, out_shapes):
    grid = _NPAD // _R
    return body, grid


_FULL = lambda *shape: pl.BlockSpec(shape, lambda i: (0,) * len(shape))


def _prep_call(body, x_first, extras):
    grid = _NPAD // _R
    in_specs = [pl.BlockSpec((_R, x_first.shape[1]), lambda i: (i, 0))]
    in_specs += [_FULL(*e.shape) for e in extras]
    out = pl.pallas_call(
        body,
        grid=(grid,),
        in_specs=in_specs,
        out_specs=[
            pl.BlockSpec((_R, _SROW), lambda i: (i, 0)),
            pl.BlockSpec((_R, 16), lambda i: (i, 0)),
            pl.BlockSpec((_R, _SROW), lambda i: (i, 0)),
        ],
        out_shape=[
            jax.ShapeDtypeStruct((_NPAD, _SROW), jnp.float32),
            jax.ShapeDtypeStruct((_NPAD, 16), jnp.float32),
            jax.ShapeDtypeStruct((_NPAD, _SROW), jnp.float32),
        ],
    )(x_first, *extras)
    return out


def kernel(atom_features, bond_features, edge_index, W_atom, b_atom, gat_W,
           att_src, att_dst, gat_b, bn_g, bn_b, W1, b1, W2, b2):
    n = atom_features.shape[0]
    nf = float(n)
    e = edge_index.shape[1]

    atom_p = jnp.pad(atom_features, ((0, _NPAD - n), (0, 128 - atom_features.shape[1])))
    wa_p = jnp.pad(W_atom, ((0, 128 - W_atom.shape[0]), (0, 0)))
    src = jnp.pad(edge_index[0], (0, _EPAD - e))
    dst = jnp.pad(edge_index[1], (0, _EPAD - e), constant_values=0x3F000000)
    bsrc, bdst, cnts = _sc_bucket(src, dst)

    rep = jnp.repeat(jnp.eye(8, dtype=jnp.float32), _HC, axis=1)
    hid_ids = jnp.repeat(jnp.arange(8), _HC)
    cols = jnp.arange(128)

    def make_ab(l):
        ab = jnp.zeros((128, 16), jnp.float32)
        ab = ab.at[cols, hid_ids].set(att_src[l].reshape(128))
        ab = ab.at[cols, 8 + hid_ids].set(att_dst[l].reshape(128))
        return ab

    s_tab, ad_tab, acc_init = _prep_call(
        _prep0_body, atom_p,
        [wa_p, b_atom[None, :], gat_W[0], make_ab(0), rep])

    grid = _NPAD // _R
    x = None
    sums = None
    for l in range(_L):
        acc = _sc_aggregate(bsrc, bdst, cnts, s_tab, ad_tab, acc_init)
        x, sums = pl.pallas_call(
            functools.partial(_fin_body, nf),
            grid=(grid,),
            in_specs=[pl.BlockSpec((_R, _SROW), lambda i: (i, 0)),
                      _FULL(8, 128), _FULL(1, 128)],
            out_specs=[pl.BlockSpec((_R, 128), lambda i: (i, 0)),
                       _FULL(8, 128)],
            out_shape=[jax.ShapeDtypeStruct((_NPAD, 128), jnp.float32),
                       jax.ShapeDtypeStruct((8, 128), jnp.float32)],
        )(acc, rep, gat_b[l][None, :])
        if l < _L - 1:
            s_tab, ad_tab, acc_init = _prep_call(
                functools.partial(_prepl_body, nf), x,
                [sums, bn_g[l][None, :], bn_b[l][None, :],
                 gat_W[l + 1], make_ab(l + 1), rep])

    ysum = pl.pallas_call(
        functools.partial(_colsum_body, nf),
        grid=(grid,),
        in_specs=[pl.BlockSpec((_R, 128), lambda i: (i, 0)),
                  _FULL(8, 128), _FULL(1, 128), _FULL(1, 128)],
        out_specs=_FULL(8, 128),
        out_shape=jax.ShapeDtypeStruct((8, 128), jnp.float32),
    )(x, sums, bn_g[_L - 1][None, :], bn_b[_L - 1][None, :])

    out = pl.pallas_call(
        functools.partial(_head_body, nf),
        grid=(1,),
        in_specs=[_FULL(8, 128), _FULL(*W1.shape), _FULL(1, W1.shape[1]),
                  _FULL(*W2.shape), _FULL(1, W2.shape[1])],
        out_specs=_FULL(1, W2.shape[1]),
        out_shape=jax.ShapeDtypeStruct((1, W2.shape[1]), jnp.float32),
    )(ysum, W1, b1[None, :], W2, b2[None, :])
    return out


# speedup vs baseline: 1.2934x; 1.0020x over previous
import functools

import jax
import jax.numpy as jnp
from jax import lax
from jax.experimental import pallas as pl
from jax.experimental.pallas import tpu as pltpu
from jax.experimental.pallas import tpu_sc as plsc

_HEADS = 8
_HC = 16
_HID = 128
_L = 3

_R = 512
_CHUNKS = 6
_CH = 8448
_NPAD = _CH * _CHUNKS
_RPT = _CH // 16
_B = 128
_SROW = 144


_EPT = 26624
_EPAD = 32 * _EPT
_STRIDE = 28032
_STG = 1312


def _f16(v):
    return jnp.full((16,), v, jnp.int32)


def _bucket_body(src_hbm, dst_hbm, bsrc_hbm, bdst_hbm, cnt_hbm,
                 src_s, dst_s, sstage, dstage, cvm):
    cid = lax.axis_index("c")
    tid = lax.axis_index("s")
    w = cid * 16 + tid
    ebase = w * _EPT
    iota = lax.broadcasted_iota(jnp.int32, (16,), 0)
    zi = jnp.zeros((16,), jnp.int32)
    for c in range(_CHUNKS):
        for j in range(_STG // 16):
            plsc.store_scatter(sstage, [_f16(c), iota + j * 16], zi)
            plsc.store_scatter(dstage, [_f16(c), iota + j * 16], zi)

    def macro(mi, carry):
        pltpu.sync_copy(src_hbm.at[pl.ds(pl.multiple_of(ebase + mi * 2048, 8), 2048)], src_s)
        pltpu.sync_copy(dst_hbm.at[pl.ds(pl.multiple_of(ebase + mi * 2048, 8), 2048)], dst_s)

        def grp(g, c2):
            rows = iota + g * 16
            s16 = plsc.load_gather(src_s, [rows])
            d16 = plsc.load_gather(dst_s, [rows])
            out = []
            for c in range(_CHUNKS):
                voff, hoff = c2[2 * c], c2[2 * c + 1]
                m = (d16 >= c * _CH) & (d16 < (c + 1) * _CH)
                mi32 = jnp.where(m, 1, 0)
                pos = voff + plsc.cumsum(mi32) - mi32
                plsc.store_scatter(sstage, [_f16(c), pos], s16, mask=m)
                plsc.store_scatter(dstage, [_f16(c), pos], d16, mask=m)
                voff = voff + jnp.sum(mi32)

                def fl(vh):
                    v, h = vh
                    off = (w * _CHUNKS + c) * _STRIDE + h
                    pltpu.sync_copy(sstage.at[c, pl.ds(0, 1024)],
                                    bsrc_hbm.at[pl.ds(pl.multiple_of(off, 8), 1024)])
                    pltpu.sync_copy(dstage.at[c, pl.ds(0, 1024)],
                                    bdst_hbm.at[pl.ds(pl.multiple_of(off, 8), 1024)])
                    tv = plsc.load_gather(sstage, [_f16(c), iota + 1024])
                    plsc.store_scatter(sstage, [_f16(c), iota], tv)
                    tv = plsc.load_gather(dstage, [_f16(c), iota + 1024])
                    plsc.store_scatter(dstage, [_f16(c), iota], tv)
                    return (v - 1024, h + 1024)

                voff, hoff = lax.cond(voff >= 1024, fl, lambda vh: vh,
                                      (voff, hoff))
                out += [voff, hoff]
            return tuple(out)

        return lax.fori_loop(0, 128, grp, carry)

    carry = lax.fori_loop(0, _EPT // 2048, macro, (jnp.int32(0),) * (2 * _CHUNKS))
    cvec = jnp.zeros((16,), jnp.int32)
    for c in range(_CHUNKS):
        voff, hoff = carry[2 * c], carry[2 * c + 1]
        off = (w * _CHUNKS + c) * _STRIDE + hoff
        pltpu.sync_copy(sstage.at[c, pl.ds(0, 1312)],
                        bsrc_hbm.at[pl.ds(pl.multiple_of(off, 8), 1312)])
        pltpu.sync_copy(dstage.at[c, pl.ds(0, 1312)],
                        bdst_hbm.at[pl.ds(pl.multiple_of(off, 8), 1312)])
        cvec = jnp.where(iota == c, voff + hoff, cvec)
    cvm[...] = cvec
    pltpu.sync_copy(cvm, cnt_hbm.at[pl.ds(pl.multiple_of(w * 16, 8), 16)])


def _sc_bucket(src, dst):
    run = pl.kernel(
        _bucket_body,
        out_type=[jax.ShapeDtypeStruct((32 * _CHUNKS * _STRIDE,), jnp.int32),
                  jax.ShapeDtypeStruct((32 * _CHUNKS * _STRIDE,), jnp.int32),
                  jax.ShapeDtypeStruct((512,), jnp.int32)],
        mesh=plsc.VectorSubcoreMesh(core_axis_name="c", subcore_axis_name="s"),
        compiler_params=pltpu.CompilerParams(use_tc_tiling_on_sc=False,
                                             needs_layout_passes=False),
        scratch_types=[
            pltpu.VMEM((2048,), jnp.int32),
            pltpu.VMEM((2048,), jnp.int32),
            pltpu.VMEM((_CHUNKS, _STG), jnp.int32),
            pltpu.VMEM((_CHUNKS, _STG), jnp.int32),
            pltpu.VMEM((16,), jnp.int32),
        ],
    )
    return run(src, dst)


def _sc_body(bsrc_hbm, bdst_hbm, cnt_hbm, s_hbm, ad_hbm, accinit_hbm, out_hbm,
             cnts_vm,
             srcb_a, dstb_a, dreli_a, svm_a, advm_a,
             srcb_b, dstb_b, dreli_b, svm_b, advm_b,
             acc_sh, sem_ea, sem_eb, sem_ga, sem_gb, sem_sa, sem_sb):
    cid = lax.axis_index("c")
    tid = lax.axis_index("s")
    iota = lax.broadcasted_iota(jnp.int32, (16,), 0)
    pltpu.sync_copy(cnt_hbm, cnts_vm)

    def payload(svm, advm):
        def edge2(e2, c2):
            colw = iota + 128
            rvs, wvs = [], []
            for u in range(2):
                rowv = iota * 0 + (e2 * 2 + u)
                as16 = plsc.load_gather(svm, [rowv, colw])
                ad16 = plsc.load_gather(advm, [rowv, iota])
                al = as16 + ad16
                al = jnp.where(al >= 0, al, 0.2 * al)
                w16 = jnp.exp(al)
                plsc.store_scatter(svm, [rowv, colw], w16)
                rvs.append(rowv)
                wvs.append(w16)
            for h in range(_HEADS):
                cols = iota + h * _HC
                for u in range(2):
                    ws = jnp.broadcast_to(wvs[u][h], (16,))
                    xv = plsc.load_gather(svm, [rvs[u], cols])
                    plsc.store_scatter(svm, [rvs[u], cols], xv * ws)
            return c2

        lax.fori_loop(0, _B // 2, edge2, 0)

    def chunk_step(k, kcarry):
        chunk = cid * (_CHUNKS // 2) + k
        base = chunk * _CH
        pltpu.sync_copy(accinit_hbm.at[pl.ds(base + tid * _RPT, _RPT)],
                        acc_sh.at[pl.ds(tid * _RPT, _RPT)])
        plsc.subcore_barrier()

        for wi in range(2):
            w = tid + wi * 16
            rowoff = (w * _CHUNKS + chunk) * _STRIDE
            cv = cnts_vm[pl.ds(pl.multiple_of(w * 16, 8), 16)]
            cnt = jnp.sum(jnp.where(iota == chunk, cv, 0))
            nblk = (cnt + _B - 1) // _B
            nstep = (nblk + 1) // 2

            def eload(bi, srcb, dstb, sem):
                off = pl.multiple_of(rowoff + bi * _B, 8)
                pltpu.async_copy(bsrc_hbm.at[pl.ds(off, _B)], srcb, sem)
                pltpu.async_copy(bdst_hbm.at[pl.ds(off, _B)], dstb, sem)

            def ewait(srcb, dstb, sem):
                pltpu.make_async_copy(
                    bsrc_hbm.at[pl.ds(0, _B)], srcb, sem).wait()
                pltpu.make_async_copy(
                    bdst_hbm.at[pl.ds(0, _B)], dstb, sem).wait()

            def half(j, bi, srcb, dstb, dreli, svm, advm,
                     sem_e, sem_g, sem_s, pf_bi, srcb_o, dstb_o, sem_eo):
                ewait(srcb, dstb, sem_e)

                @pl.when(j > 0)
                def _():
                    pltpu.make_async_copy(svm, acc_sh.at[dreli], sem_s).wait()

                gs = pltpu.async_copy(s_hbm.at[srcb], svm, sem_g)
                ga = pltpu.async_copy(ad_hbm.at[dstb], advm, sem_g)
                eload(pf_bi, srcb_o, dstb_o, sem_eo)
                for g in range(_B // 16):
                    d16 = dstb[pl.ds(g * 16, 16)]
                    pos = bi * _B + g * 16 + iota
                    dreli[pl.ds(g * 16, 16)] = jnp.where(pos < cnt,
                                                         d16 - base, _CH)
                gs.wait()
                ga.wait()
                payload(svm, advm)
                pltpu.async_copy(svm, acc_sh.at[dreli], sem_s, add=True)

            eload(0, srcb_a, dstb_a, sem_ea)

            def body(j, carry):
                half(j, 2 * j, srcb_a, dstb_a, dreli_a, svm_a, advm_a,
                     sem_ea, sem_ga, sem_sa,
                     2 * j + 1, srcb_b, dstb_b, sem_eb)
                half(j, 2 * j + 1, srcb_b, dstb_b, dreli_b, svm_b, advm_b,
                     sem_eb, sem_gb, sem_sb,
                     2 * j + 2, srcb_a, dstb_a, sem_ea)
                return carry

            lax.fori_loop(0, nstep, body, 0)
            ewait(srcb_a, dstb_a, sem_ea)

            @pl.when(nstep > 0)
            def _():
                pltpu.make_async_copy(svm_a, acc_sh.at[dreli_a], sem_sa).wait()
                pltpu.make_async_copy(svm_b, acc_sh.at[dreli_b], sem_sb).wait()

        plsc.subcore_barrier()
        pltpu.sync_copy(acc_sh.at[pl.ds(tid * _RPT, _RPT)],
                        out_hbm.at[pl.ds(base + tid * _RPT, _RPT)])
        plsc.subcore_barrier()
        return kcarry

    lax.fori_loop(0, _CHUNKS // 2, chunk_step, 0)


def _sc_aggregate(bsrc, bdst, cnts, s_tab, ad_tab, acc_init):
    run = pl.kernel(
        _sc_body,
        out_type=jax.ShapeDtypeStruct((_NPAD, _SROW), jnp.float32),
        mesh=plsc.VectorSubcoreMesh(core_axis_name="c", subcore_axis_name="s"),
        compiler_params=pltpu.CompilerParams(use_tc_tiling_on_sc=False,
                                             needs_layout_passes=False),
        scratch_types=(
            [pltpu.VMEM((512,), jnp.int32)]
            + 2 * [pltpu.VMEM((_B,), jnp.int32),
                   pltpu.VMEM((_B,), jnp.int32),
                   pltpu.VMEM((_B,), jnp.int32),
                   pltpu.VMEM((_B, _SROW), jnp.float32),
                   pltpu.VMEM((_B, 16), jnp.float32)]
            + [pltpu.VMEM_SHARED((_CH + 16, _SROW), jnp.float32)]
            + 6 * [pltpu.SemaphoreType.DMA]
        ),
    )
    return run(bsrc, bdst, cnts, s_tab, ad_tab, acc_init)


def _prep_common(y, w_ref, ab_ref, rep_ref, s_ref, ad_ref, acc_ref):
    xp = jnp.dot(y, w_ref[...], preferred_element_type=jnp.float32)
    both = jnp.dot(xp, ab_ref[...], preferred_element_type=jnp.float32)
    a_s = both[:, :8]
    a_d = both[:, 8:]
    t = a_s + a_d
    w_self = jnp.exp(jnp.where(t >= 0, t, 0.2 * t))
    wrep = jnp.dot(w_self, rep_ref[...], preferred_element_type=jnp.float32)
    z8 = jnp.zeros((y.shape[0], 8), jnp.float32)
    s_ref[...] = jnp.concatenate([xp, a_s, z8], axis=1)
    ad_ref[...] = jnp.concatenate([a_d, z8], axis=1)
    acc_ref[...] = jnp.concatenate([xp * wrep, w_self, z8], axis=1)


def _prep0_body(atom_ref, wa_ref, ba_ref, w_ref, ab_ref, rep_ref,
                s_ref, ad_ref, acc_ref):
    y = jnp.dot(atom_ref[...], wa_ref[...],
                preferred_element_type=jnp.float32) + ba_ref[...]
    _prep_common(y, w_ref, ab_ref, rep_ref, s_ref, ad_ref, acc_ref)


def _prepl_body(n, x_ref, sums_ref, bng_ref, bnb_ref, w_ref, ab_ref, rep_ref,
                s_ref, ad_ref, acc_ref):
    mu = sums_ref[0:1, :] / n
    var = sums_ref[1:2, :] / n - mu * mu
    rstd = lax.rsqrt(var + 1e-5)
    y = (x_ref[...] - mu) * rstd * bng_ref[...] + bnb_ref[...]
    y = jnp.maximum(y, 0.0)
    _prep_common(y, w_ref, ab_ref, rep_ref, s_ref, ad_ref, acc_ref)


def _fin_body(n, acc_ref, rep_ref, gatb_ref, x_ref, sums_ref):
    acc = acc_ref[...]
    dn = jnp.dot(acc[:, 128:136], rep_ref[...],
                 preferred_element_type=jnp.float32)
    x = acc[:, :128] / dn + gatb_ref[...]
    x_ref[...] = x
    rows = pl.program_id(0) * _R + lax.broadcasted_iota(jnp.int32, (_R, 1), 0)
    xm = jnp.where(rows < n, x, 0.0)
    blk = jnp.concatenate(
        [jnp.sum(xm, axis=0, keepdims=True),
         jnp.sum(xm * xm, axis=0, keepdims=True),
         jnp.zeros((6, 128), jnp.float32)], axis=0)

    @pl.when(pl.program_id(0) == 0)
    def _():
        sums_ref[...] = blk

    @pl.when(pl.program_id(0) != 0)
    def _():
        sums_ref[...] += blk


def _colsum_body(n, x_ref, sums_ref, bng_ref, bnb_ref, ysum_ref):
    mu = sums_ref[0:1, :] / n
    var = sums_ref[1:2, :] / n - mu * mu
    rstd = lax.rsqrt(var + 1e-5)
    y = (x_ref[...] - mu) * rstd * bng_ref[...] + bnb_ref[...]
    y = jnp.maximum(y, 0.0)
    rows = pl.program_id(0) * _R + lax.broadcasted_iota(jnp.int32, (_R, 1), 0)
    ym = jnp.where(rows < n, y, 0.0)
    blk = jnp.concatenate(
        [jnp.sum(ym, axis=0, keepdims=True),
         jnp.zeros((7, 128), jnp.float32)], axis=0)

    @pl.when(pl.program_id(0) == 0)
    def _():
        ysum_ref[...] = blk

    @pl.when(pl.program_id(0) != 0)
    def _():
        ysum_ref[...] += blk


def _head_body(n, ysum_ref, w1_ref, b1_ref, w2_ref, b2_ref, o_ref):
    mean = ysum_ref[0:1, :] / n
    h = jnp.maximum(jnp.dot(mean, w1_ref[...],
                            preferred_element_type=jnp.float32) + b1_ref[...],
                    0.0)
    o_ref[...] = jnp.dot(h, w2_ref[...],
                         preferred_element_type=jnp.float32) + b2_ref[...]


def _row_blocked_call(body, n_extra_in, out_specs, out_shapes):
    grid = _NPAD // _R
    return body, grid


_FULL = lambda *shape: pl.BlockSpec(shape, lambda i: (0,) * len(shape))


def _prep_call(body, x_first, extras):
    grid = _NPAD // _R
    in_specs = [pl.BlockSpec((_R, x_first.shape[1]), lambda i: (i, 0))]
    in_specs += [_FULL(*e.shape) for e in extras]
    out = pl.pallas_call(
        body,
        grid=(grid,),
        in_specs=in_specs,
        out_specs=[
            pl.BlockSpec((_R, _SROW), lambda i: (i, 0)),
            pl.BlockSpec((_R, 16), lambda i: (i, 0)),
            pl.BlockSpec((_R, _SROW), lambda i: (i, 0)),
        ],
        out_shape=[
            jax.ShapeDtypeStruct((_NPAD, _SROW), jnp.float32),
            jax.ShapeDtypeStruct((_NPAD, 16), jnp.float32),
            jax.ShapeDtypeStruct((_NPAD, _SROW), jnp.float32),
        ],
    )(x_first, *extras)
    return out


def kernel(atom_features, bond_features, edge_index, W_atom, b_atom, gat_W,
           att_src, att_dst, gat_b, bn_g, bn_b, W1, b1, W2, b2):
    n = atom_features.shape[0]
    nf = float(n)
    e = edge_index.shape[1]

    atom_p = jnp.pad(atom_features, ((0, _NPAD - n), (0, 128 - atom_features.shape[1])))
    wa_p = jnp.pad(W_atom, ((0, 128 - W_atom.shape[0]), (0, 0)))
    src = jnp.pad(edge_index[0], (0, _EPAD - e))
    dst = jnp.pad(edge_index[1], (0, _EPAD - e), constant_values=0x3F000000)
    bsrc, bdst, cnts = _sc_bucket(src, dst)

    rep = jnp.repeat(jnp.eye(8, dtype=jnp.float32), _HC, axis=1)
    hid_ids = jnp.repeat(jnp.arange(8), _HC)
    cols = jnp.arange(128)

    def make_ab(l):
        ab = jnp.zeros((128, 16), jnp.float32)
        ab = ab.at[cols, hid_ids].set(att_src[l].reshape(128))
        ab = ab.at[cols, 8 + hid_ids].set(att_dst[l].reshape(128))
        return ab

    s_tab, ad_tab, acc_init = _prep_call(
        _prep0_body, atom_p,
        [wa_p, b_atom[None, :], gat_W[0], make_ab(0), rep])

    grid = _NPAD // _R
    x = None
    sums = None
    for l in range(_L):
        acc = _sc_aggregate(bsrc, bdst, cnts, s_tab, ad_tab, acc_init)
        x, sums = pl.pallas_call(
            functools.partial(_fin_body, nf),
            grid=(grid,),
            in_specs=[pl.BlockSpec((_R, _SROW), lambda i: (i, 0)),
                      _FULL(8, 128), _FULL(1, 128)],
            out_specs=[pl.BlockSpec((_R, 128), lambda i: (i, 0)),
                       _FULL(8, 128)],
            out_shape=[jax.ShapeDtypeStruct((_NPAD, 128), jnp.float32),
                       jax.ShapeDtypeStruct((8, 128), jnp.float32)],
        )(acc, rep, gat_b[l][None, :])
        if l < _L - 1:
            s_tab, ad_tab, acc_init = _prep_call(
                functools.partial(_prepl_body, nf), x,
                [sums, bn_g[l][None, :], bn_b[l][None, :],
                 gat_W[l + 1], make_ab(l + 1), rep])

    ysum = pl.pallas_call(
        functools.partial(_colsum_body, nf),
        grid=(grid,),
        in_specs=[pl.BlockSpec((_R, 128), lambda i: (i, 0)),
                  _FULL(8, 128), _FULL(1, 128), _FULL(1, 128)],
        out_specs=_FULL(8, 128),
        out_shape=jax.ShapeDtypeStruct((8, 128), jnp.float32),
    )(x, sums, bn_g[_L - 1][None, :], bn_b[_L - 1][None, :])

    out = pl.pallas_call(
        functools.partial(_head_body, nf),
        grid=(1,),
        in_specs=[_FULL(8, 128), _FULL(*W1.shape), _FULL(1, W1.shape[1]),
                  _FULL(*W2.shape), _FULL(1, W2.shape[1])],
        out_specs=_FULL(1, W2.shape[1]),
        out_shape=jax.ShapeDtypeStruct((1, W2.shape[1]), jnp.float32),
    )(ysum, W1, b1[None, :], W2, b2[None, :])
    return out


# speedup vs baseline: 2.8585x; 2.2101x over previous
import functools

import jax
import jax.numpy as jnp
from jax import lax
from jax.experimental import pallas as pl
from jax.experimental.pallas import tpu as pltpu
from jax.experimental.pallas import tpu_sc as plsc

_HEADS = 8
_HC = 16
_HID = 128
_L = 3

_R = 512
_CHUNKS = 6
_CH = 8448
_NPAD = _CH * _CHUNKS
_RPT = _CH // 16
_B = 128
_SROW = 144


_EPT = 26624
_EPAD = 32 * _EPT
_STRIDE = 28032
_STG = 1312


def _f16(v):
    return jnp.full((16,), v, jnp.int32)


def _bucket_body(src_hbm, dst_hbm, bsrc_hbm, bdst_hbm, cnt_hbm,
                 src_s, dst_s, sstage, dstage, cvm):
    cid = lax.axis_index("c")
    tid = lax.axis_index("s")
    w = cid * 16 + tid
    ebase = w * _EPT
    iota = lax.broadcasted_iota(jnp.int32, (16,), 0)
    zi = jnp.zeros((16,), jnp.int32)
    for c in range(_CHUNKS):
        for j in range(_STG // 16):
            plsc.store_scatter(sstage, [_f16(c), iota + j * 16], zi)
            plsc.store_scatter(dstage, [_f16(c), iota + j * 16], zi)

    def macro(mi, carry):
        pltpu.sync_copy(src_hbm.at[pl.ds(pl.multiple_of(ebase + mi * 2048, 8), 2048)], src_s)
        pltpu.sync_copy(dst_hbm.at[pl.ds(pl.multiple_of(ebase + mi * 2048, 8), 2048)], dst_s)

        def grp(g, c2):
            rows = iota + g * 16
            s16 = plsc.load_gather(src_s, [rows])
            d16 = plsc.load_gather(dst_s, [rows])
            out = []
            for c in range(_CHUNKS):
                voff, hoff = c2[2 * c], c2[2 * c + 1]
                m = (d16 >= c * _CH) & (d16 < (c + 1) * _CH)
                mi32 = jnp.where(m, 1, 0)
                pos = voff + plsc.cumsum(mi32) - mi32
                plsc.store_scatter(sstage, [_f16(c), pos], s16, mask=m)
                plsc.store_scatter(dstage, [_f16(c), pos], d16, mask=m)
                voff = voff + jnp.sum(mi32)

                def fl(vh):
                    v, h = vh
                    off = (w * _CHUNKS + c) * _STRIDE + h
                    pltpu.sync_copy(sstage.at[c, pl.ds(0, 1024)],
                                    bsrc_hbm.at[pl.ds(pl.multiple_of(off, 8), 1024)])
                    pltpu.sync_copy(dstage.at[c, pl.ds(0, 1024)],
                                    bdst_hbm.at[pl.ds(pl.multiple_of(off, 8), 1024)])
                    tv = plsc.load_gather(sstage, [_f16(c), iota + 1024])
                    plsc.store_scatter(sstage, [_f16(c), iota], tv)
                    tv = plsc.load_gather(dstage, [_f16(c), iota + 1024])
                    plsc.store_scatter(dstage, [_f16(c), iota], tv)
                    return (v - 1024, h + 1024)

                voff, hoff = lax.cond(voff >= 1024, fl, lambda vh: vh,
                                      (voff, hoff))
                out += [voff, hoff]
            return tuple(out)

        return lax.fori_loop(0, 128, grp, carry)

    carry = lax.fori_loop(0, _EPT // 2048, macro, (jnp.int32(0),) * (2 * _CHUNKS))
    cvec = jnp.zeros((16,), jnp.int32)
    for c in range(_CHUNKS):
        voff, hoff = carry[2 * c], carry[2 * c + 1]
        off = (w * _CHUNKS + c) * _STRIDE + hoff
        pltpu.sync_copy(sstage.at[c, pl.ds(0, 1312)],
                        bsrc_hbm.at[pl.ds(pl.multiple_of(off, 8), 1312)])
        pltpu.sync_copy(dstage.at[c, pl.ds(0, 1312)],
                        bdst_hbm.at[pl.ds(pl.multiple_of(off, 8), 1312)])
        cvec = jnp.where(iota == c, voff + hoff, cvec)
    cvm[...] = cvec
    pltpu.sync_copy(cvm, cnt_hbm.at[pl.ds(pl.multiple_of(w * 16, 8), 16)])


def _sc_bucket(src, dst):
    run = pl.kernel(
        _bucket_body,
        out_type=[jax.ShapeDtypeStruct((32 * _CHUNKS * _STRIDE,), jnp.int32),
                  jax.ShapeDtypeStruct((32 * _CHUNKS * _STRIDE,), jnp.int32),
                  jax.ShapeDtypeStruct((512,), jnp.int32)],
        mesh=plsc.VectorSubcoreMesh(core_axis_name="c", subcore_axis_name="s"),
        compiler_params=pltpu.CompilerParams(use_tc_tiling_on_sc=False,
                                             needs_layout_passes=False),
        scratch_types=[
            pltpu.VMEM((2048,), jnp.int32),
            pltpu.VMEM((2048,), jnp.int32),
            pltpu.VMEM((_CHUNKS, _STG), jnp.int32),
            pltpu.VMEM((_CHUNKS, _STG), jnp.int32),
            pltpu.VMEM((16,), jnp.int32),
        ],
    )
    return run(src, dst)


def _sc_body(bsrc_hbm, bdst_hbm, cnt_hbm, s_hbm, ad_hbm, accinit_hbm, out_hbm,
             cnts_vm,
             srcb_a, dstb_a, dreli_a, svm_a, advm_a,
             srcb_b, dstb_b, dreli_b, svm_b, advm_b,
             acc_sh, sem_ea, sem_eb, sem_ga, sem_gb, sem_sa, sem_sb):
    cid = lax.axis_index("c")
    tid = lax.axis_index("s")
    iota = lax.broadcasted_iota(jnp.int32, (16,), 0)
    pltpu.sync_copy(cnt_hbm, cnts_vm)

    def payload(svm, advm):
        def edge2(e2, c2):
            colw = iota + 128
            rvs, wvs = [], []
            for u in range(2):
                rowv = iota * 0 + (e2 * 2 + u)
                as16 = plsc.load_gather(svm, [rowv, colw])
                ad16 = plsc.load_gather(advm, [rowv, iota])
                al = as16 + ad16
                al = jnp.where(al >= 0, al, 0.2 * al)
                w16 = jnp.exp(al)
                plsc.store_scatter(svm, [rowv, colw], w16)
                rvs.append(rowv)
                wvs.append(w16)
            for h in range(_HEADS):
                cols = iota + h * _HC
                for u in range(2):
                    ws = jnp.broadcast_to(wvs[u][h], (16,))
                    xv = plsc.load_gather(svm, [rvs[u], cols])
                    plsc.store_scatter(svm, [rvs[u], cols], xv * ws)
            return c2

        plsc.parallel_loop(0, _B // 2, carry=jnp.int32(0))(edge2)

    def chunk_step(k, kcarry):
        chunk = cid * (_CHUNKS // 2) + k
        base = chunk * _CH
        pltpu.sync_copy(accinit_hbm.at[pl.ds(base + tid * _RPT, _RPT)],
                        acc_sh.at[pl.ds(tid * _RPT, _RPT)])
        plsc.subcore_barrier()

        for wi in range(2):
            w = tid + wi * 16
            rowoff = (w * _CHUNKS + chunk) * _STRIDE
            cv = cnts_vm[pl.ds(pl.multiple_of(w * 16, 8), 16)]
            cnt = jnp.sum(jnp.where(iota == chunk, cv, 0))
            nblk = (cnt + _B - 1) // _B
            nstep = (nblk + 1) // 2

            def eload(bi, srcb, dstb, sem):
                off = pl.multiple_of(rowoff + bi * _B, 8)
                pltpu.async_copy(bsrc_hbm.at[pl.ds(off, _B)], srcb, sem)
                pltpu.async_copy(bdst_hbm.at[pl.ds(off, _B)], dstb, sem)

            def ewait(srcb, dstb, sem):
                pltpu.make_async_copy(
                    bsrc_hbm.at[pl.ds(0, _B)], srcb, sem).wait()
                pltpu.make_async_copy(
                    bdst_hbm.at[pl.ds(0, _B)], dstb, sem).wait()

            def half(j, bi, srcb, dstb, dreli, svm, advm,
                     sem_e, sem_g, sem_s, pf_bi, srcb_o, dstb_o, sem_eo):
                ewait(srcb, dstb, sem_e)

                @pl.when(j > 0)
                def _():
                    pltpu.make_async_copy(svm, acc_sh.at[dreli], sem_s).wait()

                gs = pltpu.async_copy(s_hbm.at[srcb], svm, sem_g)
                ga = pltpu.async_copy(ad_hbm.at[dstb], advm, sem_g)
                eload(pf_bi, srcb_o, dstb_o, sem_eo)
                for g in range(_B // 16):
                    d16 = dstb[pl.ds(g * 16, 16)]
                    pos = bi * _B + g * 16 + iota
                    dreli[pl.ds(g * 16, 16)] = jnp.where(pos < cnt,
                                                         d16 - base, _CH)
                gs.wait()
                ga.wait()
                payload(svm, advm)
                pltpu.async_copy(svm, acc_sh.at[dreli], sem_s, add=True)

            eload(0, srcb_a, dstb_a, sem_ea)

            def body(j, carry):
                half(j, 2 * j, srcb_a, dstb_a, dreli_a, svm_a, advm_a,
                     sem_ea, sem_ga, sem_sa,
                     2 * j + 1, srcb_b, dstb_b, sem_eb)
                half(j, 2 * j + 1, srcb_b, dstb_b, dreli_b, svm_b, advm_b,
                     sem_eb, sem_gb, sem_sb,
                     2 * j + 2, srcb_a, dstb_a, sem_ea)
                return carry

            lax.fori_loop(0, nstep, body, 0)
            ewait(srcb_a, dstb_a, sem_ea)

            @pl.when(nstep > 0)
            def _():
                pltpu.make_async_copy(svm_a, acc_sh.at[dreli_a], sem_sa).wait()
                pltpu.make_async_copy(svm_b, acc_sh.at[dreli_b], sem_sb).wait()

        plsc.subcore_barrier()
        pltpu.sync_copy(acc_sh.at[pl.ds(tid * _RPT, _RPT)],
                        out_hbm.at[pl.ds(base + tid * _RPT, _RPT)])
        plsc.subcore_barrier()
        return kcarry

    lax.fori_loop(0, _CHUNKS // 2, chunk_step, 0)


def _sc_aggregate(bsrc, bdst, cnts, s_tab, ad_tab, acc_init):
    run = pl.kernel(
        _sc_body,
        out_type=jax.ShapeDtypeStruct((_NPAD, _SROW), jnp.float32),
        mesh=plsc.VectorSubcoreMesh(core_axis_name="c", subcore_axis_name="s"),
        compiler_params=pltpu.CompilerParams(use_tc_tiling_on_sc=False,
                                             needs_layout_passes=False),
        scratch_types=(
            [pltpu.VMEM((512,), jnp.int32)]
            + 2 * [pltpu.VMEM((_B,), jnp.int32),
                   pltpu.VMEM((_B,), jnp.int32),
                   pltpu.VMEM((_B,), jnp.int32),
                   pltpu.VMEM((_B, _SROW), jnp.float32),
                   pltpu.VMEM((_B, 16), jnp.float32)]
            + [pltpu.VMEM_SHARED((_CH + 16, _SROW), jnp.float32)]
            + 6 * [pltpu.SemaphoreType.DMA]
        ),
    )
    return run(bsrc, bdst, cnts, s_tab, ad_tab, acc_init)


def _prep_common(y, w_ref, ab_ref, rep_ref, s_ref, ad_ref, acc_ref):
    xp = jnp.dot(y, w_ref[...], preferred_element_type=jnp.float32)
    both = jnp.dot(xp, ab_ref[...], preferred_element_type=jnp.float32)
    a_s = both[:, :8]
    a_d = both[:, 8:]
    t = a_s + a_d
    w_self = jnp.exp(jnp.where(t >= 0, t, 0.2 * t))
    wrep = jnp.dot(w_self, rep_ref[...], preferred_element_type=jnp.float32)
    z8 = jnp.zeros((y.shape[0], 8), jnp.float32)
    s_ref[...] = jnp.concatenate([xp, a_s, z8], axis=1)
    ad_ref[...] = jnp.concatenate([a_d, z8], axis=1)
    acc_ref[...] = jnp.concatenate([xp * wrep, w_self, z8], axis=1)


def _prep0_body(atom_ref, wa_ref, ba_ref, w_ref, ab_ref, rep_ref,
                s_ref, ad_ref, acc_ref):
    y = jnp.dot(atom_ref[...], wa_ref[...],
                preferred_element_type=jnp.float32) + ba_ref[...]
    _prep_common(y, w_ref, ab_ref, rep_ref, s_ref, ad_ref, acc_ref)


def _prepl_body(n, x_ref, sums_ref, bng_ref, bnb_ref, w_ref, ab_ref, rep_ref,
                s_ref, ad_ref, acc_ref):
    mu = sums_ref[0:1, :] / n
    var = sums_ref[1:2, :] / n - mu * mu
    rstd = lax.rsqrt(var + 1e-5)
    y = (x_ref[...] - mu) * rstd * bng_ref[...] + bnb_ref[...]
    y = jnp.maximum(y, 0.0)
    _prep_common(y, w_ref, ab_ref, rep_ref, s_ref, ad_ref, acc_ref)


def _fin_body(n, acc_ref, rep_ref, gatb_ref, x_ref, sums_ref):
    acc = acc_ref[...]
    dn = jnp.dot(acc[:, 128:136], rep_ref[...],
                 preferred_element_type=jnp.float32)
    x = acc[:, :128] / dn + gatb_ref[...]
    x_ref[...] = x
    rows = pl.program_id(0) * _R + lax.broadcasted_iota(jnp.int32, (_R, 1), 0)
    xm = jnp.where(rows < n, x, 0.0)
    blk = jnp.concatenate(
        [jnp.sum(xm, axis=0, keepdims=True),
         jnp.sum(xm * xm, axis=0, keepdims=True),
         jnp.zeros((6, 128), jnp.float32)], axis=0)

    @pl.when(pl.program_id(0) == 0)
    def _():
        sums_ref[...] = blk

    @pl.when(pl.program_id(0) != 0)
    def _():
        sums_ref[...] += blk


def _colsum_body(n, x_ref, sums_ref, bng_ref, bnb_ref, ysum_ref):
    mu = sums_ref[0:1, :] / n
    var = sums_ref[1:2, :] / n - mu * mu
    rstd = lax.rsqrt(var + 1e-5)
    y = (x_ref[...] - mu) * rstd * bng_ref[...] + bnb_ref[...]
    y = jnp.maximum(y, 0.0)
    rows = pl.program_id(0) * _R + lax.broadcasted_iota(jnp.int32, (_R, 1), 0)
    ym = jnp.where(rows < n, y, 0.0)
    blk = jnp.concatenate(
        [jnp.sum(ym, axis=0, keepdims=True),
         jnp.zeros((7, 128), jnp.float32)], axis=0)

    @pl.when(pl.program_id(0) == 0)
    def _():
        ysum_ref[...] = blk

    @pl.when(pl.program_id(0) != 0)
    def _():
        ysum_ref[...] += blk


def _head_body(n, ysum_ref, w1_ref, b1_ref, w2_ref, b2_ref, o_ref):
    mean = ysum_ref[0:1, :] / n
    h = jnp.maximum(jnp.dot(mean, w1_ref[...],
                            preferred_element_type=jnp.float32) + b1_ref[...],
                    0.0)
    o_ref[...] = jnp.dot(h, w2_ref[...],
                         preferred_element_type=jnp.float32) + b2_ref[...]


def _row_blocked_call(body, n_extra_in, out_specs, out_shapes):
    grid = _NPAD // _R
    return body, grid


_FULL = lambda *shape: pl.BlockSpec(shape, lambda i: (0,) * len(shape))


def _prep_call(body, x_first, extras):
    grid = _NPAD // _R
    in_specs = [pl.BlockSpec((_R, x_first.shape[1]), lambda i: (i, 0))]
    in_specs += [_FULL(*e.shape) for e in extras]
    out = pl.pallas_call(
        body,
        grid=(grid,),
        in_specs=in_specs,
        out_specs=[
            pl.BlockSpec((_R, _SROW), lambda i: (i, 0)),
            pl.BlockSpec((_R, 16), lambda i: (i, 0)),
            pl.BlockSpec((_R, _SROW), lambda i: (i, 0)),
        ],
        out_shape=[
            jax.ShapeDtypeStruct((_NPAD, _SROW), jnp.float32),
            jax.ShapeDtypeStruct((_NPAD, 16), jnp.float32),
            jax.ShapeDtypeStruct((_NPAD, _SROW), jnp.float32),
        ],
    )(x_first, *extras)
    return out


def kernel(atom_features, bond_features, edge_index, W_atom, b_atom, gat_W,
           att_src, att_dst, gat_b, bn_g, bn_b, W1, b1, W2, b2):
    n = atom_features.shape[0]
    nf = float(n)
    e = edge_index.shape[1]

    atom_p = jnp.pad(atom_features, ((0, _NPAD - n), (0, 128 - atom_features.shape[1])))
    wa_p = jnp.pad(W_atom, ((0, 128 - W_atom.shape[0]), (0, 0)))
    src = jnp.pad(edge_index[0], (0, _EPAD - e))
    dst = jnp.pad(edge_index[1], (0, _EPAD - e), constant_values=0x3F000000)
    bsrc, bdst, cnts = _sc_bucket(src, dst)

    rep = jnp.repeat(jnp.eye(8, dtype=jnp.float32), _HC, axis=1)
    hid_ids = jnp.repeat(jnp.arange(8), _HC)
    cols = jnp.arange(128)

    def make_ab(l):
        ab = jnp.zeros((128, 16), jnp.float32)
        ab = ab.at[cols, hid_ids].set(att_src[l].reshape(128))
        ab = ab.at[cols, 8 + hid_ids].set(att_dst[l].reshape(128))
        return ab

    s_tab, ad_tab, acc_init = _prep_call(
        _prep0_body, atom_p,
        [wa_p, b_atom[None, :], gat_W[0], make_ab(0), rep])

    grid = _NPAD // _R
    x = None
    sums = None
    for l in range(_L):
        acc = _sc_aggregate(bsrc, bdst, cnts, s_tab, ad_tab, acc_init)
        x, sums = pl.pallas_call(
            functools.partial(_fin_body, nf),
            grid=(grid,),
            in_specs=[pl.BlockSpec((_R, _SROW), lambda i: (i, 0)),
                      _FULL(8, 128), _FULL(1, 128)],
            out_specs=[pl.BlockSpec((_R, 128), lambda i: (i, 0)),
                       _FULL(8, 128)],
            out_shape=[jax.ShapeDtypeStruct((_NPAD, 128), jnp.float32),
                       jax.ShapeDtypeStruct((8, 128), jnp.float32)],
        )(acc, rep, gat_b[l][None, :])
        if l < _L - 1:
            s_tab, ad_tab, acc_init = _prep_call(
                functools.partial(_prepl_body, nf), x,
                [sums, bn_g[l][None, :], bn_b[l][None, :],
                 gat_W[l + 1], make_ab(l + 1), rep])

    ysum = pl.pallas_call(
        functools.partial(_colsum_body, nf),
        grid=(grid,),
        in_specs=[pl.BlockSpec((_R, 128), lambda i: (i, 0)),
                  _FULL(8, 128), _FULL(1, 128), _FULL(1, 128)],
        out_specs=_FULL(8, 128),
        out_shape=jax.ShapeDtypeStruct((8, 128), jnp.float32),
    )(x, sums, bn_g[_L - 1][None, :], bn_b[_L - 1][None, :])

    out = pl.pallas_call(
        functools.partial(_head_body, nf),
        grid=(1,),
        in_specs=[_FULL(8, 128), _FULL(*W1.shape), _FULL(1, W1.shape[1]),
                  _FULL(*W2.shape), _FULL(1, W2.shape[1])],
        out_specs=_FULL(1, W2.shape[1]),
        out_shape=jax.ShapeDtypeStruct((1, W2.shape[1]), jnp.float32),
    )(ysum, W1, b1[None, :], W2, b2[None, :])
    return out
